# Initial kernel scaffold; baseline (speedup 1.0000x reference)
import jax
import jax.numpy as jnp
from jax.experimental import pallas as pl


def kernel(x, t_emb, Wd0, We0, W):
    raise NotImplementedError("write your pallas kernel here")


# speedup vs baseline: 1.7457x; 1.7457x over previous
import jax
import jax.numpy as jnp
from jax.experimental import pallas as pl
from jax.experimental.pallas import tpu as pltpu

_B, _N, _T = 4, 512, 2048
_HID = 64
_K = 32
_AT = 128


def _adj_body(er_ref, et_ref, w_ref, out_ref):
    er = er_ref[0]
    et = et_ref[0]
    wf = w_ref[...][:1].astype(jnp.float32)
    acc = jnp.zeros((_AT, _N), jnp.float32)
    for h in range(_HID):
        p = er[:, h:h + 1] * et[h:h + 1, :]
        p = p.astype(jnp.bfloat16).astype(jnp.float32)
        acc = acc + p * wf[0, h]
    a = jnp.maximum(acc, 0.0)
    t = jnp.full((_AT, 1), jnp.inf, jnp.float32)
    for _ in range(_K):
        t = jnp.max(jnp.where(a < t, a, -jnp.inf), axis=-1, keepdims=True)
    thr = jnp.maximum(t, 0.0)
    zt = jnp.where(a >= thr, a, 0.0)
    s = jnp.sum(zt, axis=-1, keepdims=True) + 1e-5
    out_ref[0] = zt / s


def _mm_l2(t):
    mn = jnp.min(t, axis=-1, keepdims=True)
    mx = jnp.max(t, axis=-1, keepdims=True)
    t = (t - mn) / (mx - mn + 1.0)
    n = jnp.linalg.norm(t, ord=2, axis=2, keepdims=True)
    return t / jnp.maximum(n, 1e-12)


def kernel(x, t_emb, Wd0, We0, W):
    xn10 = jnp.abs(jnp.fft.rfft(x, axis=-1, norm='ortho'))
    xn10 = _mm_l2(xn10)
    te = _mm_l2(t_emb)
    xn10 = jnp.matmul(xn10, Wd0)
    xn10 = jnp.concatenate([xn10, te], axis=2)
    xn10 = jax.nn.relu(jnp.matmul(xn10, We0))
    m = jnp.mean(xn10, axis=-1, keepdims=True)
    v = jnp.var(xn10, axis=-1, keepdims=True)
    eb = (xn10 - m) / jnp.sqrt(v + 1e-08)

    et = jnp.swapaxes(eb, 1, 2)
    w16 = jnp.broadcast_to(W.reshape(1, _HID), (8, _HID)).astype(jnp.bfloat16)

    adj = pl.pallas_call(
        _adj_body,
        grid=(_B, _N // _AT),
        in_specs=[
            pl.BlockSpec((1, _AT, _HID), lambda b, r: (b, r, 0)),
            pl.BlockSpec((1, _HID, _N), lambda b, r: (b, 0, 0)),
            pl.BlockSpec((8, _HID), lambda b, r: (0, 0)),
        ],
        out_specs=pl.BlockSpec((1, _AT, _N), lambda b, r: (b, r, 0)),
        out_shape=jax.ShapeDtypeStruct((_B, _N, _N), jnp.float32),
    )(eb, et, w16)
    return adj
